# baseline (speedup 1.0000x reference)
import functools

import jax
import jax.numpy as jnp
import numpy as np
from jax import lax
from jax.experimental import pallas as pl
from jax.experimental.pallas import tpu as pltpu
from jax.experimental.pallas import tpu_sc as plsc

_K = 20
_M = 8
_C = 64
_B = 4
_N = 10000
_P = _B * _N
_R = _K * _P

_NC = 2
_NS = 16
_NW = _NC * _NS
_ROWS_W = _R // _NW
_CH = 1000
_NCHUNK = _ROWS_W // _CH


def _sc_gather_body(ft_hbm, ct_hbm, idx_hbm, fg_hbm, cg_hbm,
                    idx_v, fbuf, cbuf, fsem, csem):
    wid = lax.axis_index("s") * _NC + lax.axis_index("c")

    def body(i, carry):
        base = wid * _ROWS_W + i * _CH
        pltpu.sync_copy(idx_hbm.at[pl.ds(base, _CH)], idx_v)
        fcp = pltpu.async_copy(ft_hbm.at[idx_v], fbuf, fsem)
        ccp = pltpu.async_copy(ct_hbm.at[idx_v], cbuf, csem)
        fcp.wait()
        ccp.wait()
        pltpu.sync_copy(fbuf, fg_hbm.at[pl.ds(base, _CH)])
        pltpu.sync_copy(cbuf, cg_hbm.at[pl.ds(base, _CH)])
        return carry

    lax.fori_loop(0, _NCHUNK, body, 0)


@jax.jit
def _sc_gather(ft, ct, idx):
    mesh = plsc.VectorSubcoreMesh(core_axis_name="c", subcore_axis_name="s")
    f = pl.kernel(
        _sc_gather_body,
        out_type=[
            jax.ShapeDtypeStruct((_R, _C), jnp.float32),
            jax.ShapeDtypeStruct((_R, 8), jnp.float32),
        ],
        mesh=mesh,
        compiler_params=pltpu.CompilerParams(use_tc_tiling_on_sc=False),
        scratch_types=[
            pltpu.VMEM((_CH,), jnp.int32),
            pltpu.VMEM((_CH, _C), jnp.float32),
            pltpu.VMEM((_CH, 8), jnp.float32),
            pltpu.SemaphoreType.DMA,
            pltpu.SemaphoreType.DMA,
        ],
    )
    return f(ft, ct, idx)


_PB = 320
_GRID = _P // _PB


def _tc_body(fg_ref, cg_ref, op_ref, kp_ref, e_ref, ftres_ref, wr_ref,
             b_ref, o_ref):
    kp = kp_ref[...]
    c0 = cg_ref[0]
    pms = []
    s = jnp.zeros((_PB, _M), jnp.float32)
    for k in range(_K):
        xr = cg_ref[k] - c0
        pmk = jnp.dot(xr, kp, preferred_element_type=jnp.float32)
        pmk = jnp.maximum(pmk + op_ref[k:k + 1, :], 0.0)
        pms.append(pmk)
        s = s + pmk
    r = 1.0 / (s + 1e-6)

    e = e_ref[...]
    g = jnp.zeros((_PB, _M * _C), jnp.float32)
    for k in range(_K):
        w = pms[k] * r
        wx = jnp.dot(w, e, preferred_element_type=jnp.float32)
        fgk = fg_ref[k]
        fgt = jnp.concatenate([fgk] * _M, axis=1)
        g = g + wx * fgt
    out = jnp.dot(g, wr_ref[...], preferred_element_type=jnp.float32)
    out = out + b_ref[...]
    out = jnp.maximum(out, 0.2 * out)
    o_ref[...] = out + ftres_ref[...]


@jax.jit
def _tc_compute(fg3, cg3, one_padding, kp, e, ft, wr, bias):
    return pl.pallas_call(
        _tc_body,
        grid=(_GRID,),
        in_specs=[
            pl.BlockSpec((_K, _PB, _C), lambda i: (0, i, 0)),
            pl.BlockSpec((_K, _PB, 8), lambda i: (0, i, 0)),
            pl.BlockSpec((_K, _M), lambda i: (0, 0)),
            pl.BlockSpec((_M, _M), lambda i: (0, 0)),
            pl.BlockSpec((_M, _M * _C), lambda i: (0, 0)),
            pl.BlockSpec((_PB, _C), lambda i: (i, 0)),
            pl.BlockSpec((_M * _C, _C), lambda i: (0, 0)),
            pl.BlockSpec((1, _C), lambda i: (0, 0)),
        ],
        out_specs=pl.BlockSpec((_PB, _C), lambda i: (i, 0)),
        out_shape=jax.ShapeDtypeStruct((_P, _C), jnp.float32),
    )(fg3, cg3, one_padding, kp, e, ft, wr, bias)


def kernel(x, feature, neigh_indexs, kernels, one_padding, conv_w, conv_b):
    ft = feature.transpose(0, 2, 1).reshape(_P, _C)
    ct = jnp.pad(x.transpose(0, 2, 1).reshape(_P, 3), ((0, 0), (0, 5)))
    offs = (jnp.arange(_B, dtype=jnp.int32) * _N).reshape(_B, 1, 1)
    gidx = neigh_indexs[:, :, :_K] + offs
    idx_t = gidx.reshape(_P, _K).T.reshape(_R)

    fg, cg = _sc_gather(ft, ct, idx_t)
    fg3 = fg.reshape(_K, _P, _C)
    cg3 = cg.reshape(_K, _P, 8)

    kp = jnp.pad(kernels, ((0, 5), (0, 0)))
    wr = conv_w.reshape(_C, _C, _M).transpose(2, 1, 0).reshape(_M * _C, _C)
    e = jnp.asarray(np.kron(np.eye(_M, dtype=np.float32),
                            np.ones((1, _C), np.float32)))
    bias = conv_b.reshape(1, _C)

    out = _tc_compute(fg3, cg3, one_padding, kp, e, ft, wr, bias)
    return out.reshape(_B, _N, _C).transpose(0, 2, 1)
